# baseline (device time: 10477 ns/iter reference)
import jax
import jax.numpy as jnp
from jax import lax
from jax.experimental import pallas as pl
from jax.experimental.pallas import tpu as pltpu

NC = 4


def kernel(A, B):
    m, k = A.shape
    _, n = B.shape
    mc = m // NC

    def body(a_ref, b_ref, out_ref,
             send_q, recv_q, scale_s, scale_r,
             send_sems, recv_sems, ssend_sem, srecv_sem):
        my_x = lax.axis_index("x")
        my_y = lax.axis_index("y")
        nbr = (my_x, 1 - my_y)

        barrier_sem = pltpu.get_barrier_semaphore()
        pl.semaphore_signal(
            barrier_sem, inc=1,
            device_id=nbr, device_id_type=pl.DeviceIdType.MESH,
        )

        lane = lax.broadcasted_iota(jnp.int32, (8, 128), 1)
        scale_row = jnp.zeros((8, 128), jnp.float32)
        rdmas = []
        for c in range(NC):
            rows = pl.ds(c * mc, mc)
            partial = jnp.dot(
                a_ref[rows, :], b_ref[:, :],
                preferred_element_type=jnp.float32,
            )
            out_ref[rows, :] = partial
            s = jnp.maximum(jnp.max(jnp.abs(partial)), 1e-30)
            scale_row = scale_row + jnp.where(lane == c, s, 0.0)
            send_q[c] = jnp.round(partial * (127.0 / s)).astype(jnp.int8)
            if c == 0:
                pl.semaphore_wait(barrier_sem, 1)
            rdma = pltpu.make_async_remote_copy(
                src_ref=send_q.at[c],
                dst_ref=recv_q.at[c],
                send_sem=send_sems.at[c],
                recv_sem=recv_sems.at[c],
                device_id=nbr,
                device_id_type=pl.DeviceIdType.MESH,
            )
            rdma.start()
            rdmas.append(rdma)

        scale_s[:, :] = scale_row
        s_rdma = pltpu.make_async_remote_copy(
            src_ref=scale_s, dst_ref=scale_r,
            send_sem=ssend_sem, recv_sem=srecv_sem,
            device_id=nbr, device_id_type=pl.DeviceIdType.MESH,
        )
        s_rdma.start()

        s_rdma.wait_recv()
        for c in range(NC):
            rows = pl.ds(c * mc, mc)
            dq = scale_r[0, c] * (1.0 / 127.0)
            rdmas[c].wait_recv()
            out_ref[rows, :] = out_ref[rows, :] + recv_q[c].astype(jnp.float32) * dq
        s_rdma.wait_send()
        for c in range(NC):
            rdmas[c].wait_send()

    return pl.pallas_call(
        body,
        out_shape=jax.ShapeDtypeStruct((m, n), jnp.float32),
        in_specs=[
            pl.BlockSpec(memory_space=pltpu.VMEM),
            pl.BlockSpec(memory_space=pltpu.VMEM),
        ],
        out_specs=pl.BlockSpec(memory_space=pltpu.VMEM),
        scratch_shapes=[
            pltpu.VMEM((NC, mc, n), jnp.int8),
            pltpu.VMEM((NC, mc, n), jnp.int8),
            pltpu.VMEM((8, 128), jnp.float32),
            pltpu.VMEM((8, 128), jnp.float32),
            pltpu.SemaphoreType.DMA((NC,)),
            pltpu.SemaphoreType.DMA((NC,)),
            pltpu.SemaphoreType.DMA,
            pltpu.SemaphoreType.DMA,
        ],
        compiler_params=pltpu.CompilerParams(collective_id=0),
    )(A, B)


# device time: 10436 ns/iter; 1.0039x vs baseline; 1.0039x over previous
import os

import jax
import jax.numpy as jnp
from jax import lax
from jax.experimental import pallas as pl
from jax.experimental.pallas import tpu as pltpu

NC = int(os.environ.get("KERNEL_NC", "4"))


def kernel(A, B):
    m, k = A.shape
    _, n = B.shape
    mc = m // NC

    def body(a_ref, b_ref, out_ref,
             send_q, recv_q, scale_s, scale_r,
             send_sems, recv_sems, ssend_sem, srecv_sem):
        my_x = lax.axis_index("x")
        my_y = lax.axis_index("y")
        nbr = (my_x, 1 - my_y)

        barrier_sem = pltpu.get_barrier_semaphore()
        pl.semaphore_signal(
            barrier_sem, inc=1,
            device_id=nbr, device_id_type=pl.DeviceIdType.MESH,
        )

        for c in range(NC):
            rows = pl.ds(c * mc, mc)
            out_ref[rows, :] = jnp.dot(
                a_ref[rows, :], b_ref[:, :],
                preferred_element_type=jnp.float32,
            )

        s = jnp.maximum(jnp.max(jnp.abs(out_ref[:, :])), 1e-30)
        scale_s[:, :] = jnp.zeros((8, 128), jnp.float32) + s
        pl.semaphore_wait(barrier_sem, 1)

        s_rdma = pltpu.make_async_remote_copy(
            src_ref=scale_s, dst_ref=scale_r,
            send_sem=ssend_sem, recv_sem=srecv_sem,
            device_id=nbr, device_id_type=pl.DeviceIdType.MESH,
        )
        s_rdma.start()

        inv = 127.0 / s
        rdmas = []
        for c in range(NC):
            rows = pl.ds(c * mc, mc)
            send_q[c] = jnp.round(out_ref[rows, :] * inv).astype(jnp.int8)
            rdma = pltpu.make_async_remote_copy(
                src_ref=send_q.at[c],
                dst_ref=recv_q.at[c],
                send_sem=send_sems.at[c],
                recv_sem=recv_sems.at[c],
                device_id=nbr,
                device_id_type=pl.DeviceIdType.MESH,
            )
            rdma.start()
            rdmas.append(rdma)

        s_rdma.wait_recv()
        dq = scale_r[0, 0] * (1.0 / 127.0)
        for c in range(NC):
            rows = pl.ds(c * mc, mc)
            rdmas[c].wait_recv()
            out_ref[rows, :] = out_ref[rows, :] + recv_q[c].astype(jnp.float32) * dq
        s_rdma.wait_send()
        for c in range(NC):
            rdmas[c].wait_send()

    return pl.pallas_call(
        body,
        out_shape=jax.ShapeDtypeStruct((m, n), jnp.float32),
        in_specs=[
            pl.BlockSpec(memory_space=pltpu.VMEM),
            pl.BlockSpec(memory_space=pltpu.VMEM),
        ],
        out_specs=pl.BlockSpec(memory_space=pltpu.VMEM),
        scratch_shapes=[
            pltpu.VMEM((NC, mc, n), jnp.int8),
            pltpu.VMEM((NC, mc, n), jnp.int8),
            pltpu.VMEM((8, 128), jnp.float32),
            pltpu.VMEM((8, 128), jnp.float32),
            pltpu.SemaphoreType.DMA((NC,)),
            pltpu.SemaphoreType.DMA((NC,)),
            pltpu.SemaphoreType.DMA,
            pltpu.SemaphoreType.DMA,
        ],
        compiler_params=pltpu.CompilerParams(collective_id=0),
    )(A, B)


# device time: 10246 ns/iter; 1.0225x vs baseline; 1.0185x over previous
import jax
import jax.numpy as jnp
from jax import lax
from jax.experimental import pallas as pl
from jax.experimental.pallas import tpu as pltpu

NC = 4


def kernel(A, B):
    m, k = A.shape
    _, n = B.shape
    mc = m // NC

    def body(a_ref, b_ref, out_ref,
             send_q, recv_q, scale_s, scale_r,
             send_sems, recv_sems, ssend_sem, srecv_sem):
        my_x = lax.axis_index("x")
        my_y = lax.axis_index("y")
        nbr = (my_x, 1 - my_y)

        barrier_sem = pltpu.get_barrier_semaphore()
        pl.semaphore_signal(
            barrier_sem, inc=1,
            device_id=nbr, device_id_type=pl.DeviceIdType.MESH,
        )

        for c in range(NC):
            rows = pl.ds(c * mc, mc)
            out_ref[rows, :] = jnp.dot(
                a_ref[rows, :], b_ref[:, :],
                preferred_element_type=jnp.float32,
            )

        s = jnp.maximum(jnp.max(jnp.abs(out_ref[:, :])), 1e-30)
        scale_s[:, :] = jnp.zeros((8, 128), jnp.float32) + s
        pl.semaphore_wait(barrier_sem, 1)

        s_rdma = pltpu.make_async_remote_copy(
            src_ref=scale_s, dst_ref=scale_r,
            send_sem=ssend_sem, recv_sem=srecv_sem,
            device_id=nbr, device_id_type=pl.DeviceIdType.MESH,
        )
        s_rdma.start()

        inv = 127.0 / s
        rdmas = []
        for c in range(NC):
            rows = pl.ds(c * mc, mc)
            send_q[c] = jnp.round(out_ref[rows, :] * inv).astype(jnp.int8)
            rdma = pltpu.make_async_remote_copy(
                src_ref=send_q.at[c],
                dst_ref=recv_q.at[c],
                send_sem=send_sems.at[c],
                recv_sem=recv_sems.at[c],
                device_id=nbr,
                device_id_type=pl.DeviceIdType.MESH,
            )
            rdma.start()
            rdmas.append(rdma)

        s_rdma.wait_recv()
        dq = scale_r[0, 0] * (1.0 / 127.0)
        for c in range(NC):
            rows = pl.ds(c * mc, mc)
            rdmas[c].wait_recv()
            out_ref[rows, :] = out_ref[rows, :] + recv_q[c].astype(jnp.float32) * dq
        s_rdma.wait_send()
        for c in range(NC):
            rdmas[c].wait_send()

    return pl.pallas_call(
        body,
        out_shape=jax.ShapeDtypeStruct((m, n), jnp.float32),
        in_specs=[
            pl.BlockSpec(memory_space=pltpu.VMEM),
            pl.BlockSpec(memory_space=pltpu.VMEM),
        ],
        out_specs=pl.BlockSpec(memory_space=pltpu.VMEM),
        scratch_shapes=[
            pltpu.VMEM((NC, mc, n), jnp.int8),
            pltpu.VMEM((NC, mc, n), jnp.int8),
            pltpu.VMEM((8, 128), jnp.float32),
            pltpu.VMEM((8, 128), jnp.float32),
            pltpu.SemaphoreType.DMA((NC,)),
            pltpu.SemaphoreType.DMA((NC,)),
            pltpu.SemaphoreType.DMA,
            pltpu.SemaphoreType.DMA,
        ],
        compiler_params=pltpu.CompilerParams(collective_id=0),
    )(A, B)


# device time: 10234 ns/iter; 1.0237x vs baseline; 1.0012x over previous
import jax
import jax.numpy as jnp
from jax import lax
from jax.experimental import pallas as pl
from jax.experimental.pallas import tpu as pltpu

NC = 4


def kernel(A, B):
    m, k = A.shape
    _, n = B.shape
    mc = m // NC

    def body(a_ref, b_ref, out_ref,
             send_q, recv_q, scale_s, scale_r,
             send_sems, recv_sems, ssend_sem, srecv_sem):
        my_x = lax.axis_index("x")
        my_y = lax.axis_index("y")
        nbr = (my_x, 1 - my_y)

        barrier_sem = pltpu.get_barrier_semaphore()
        pl.semaphore_signal(
            barrier_sem, inc=1,
            device_id=nbr, device_id_type=pl.DeviceIdType.MESH,
        )

        for c in range(NC):
            rows = pl.ds(c * mc, mc)
            out_ref[rows, :] = jnp.dot(
                a_ref[rows, :].astype(jnp.bfloat16),
                b_ref[:, :].astype(jnp.bfloat16),
                preferred_element_type=jnp.float32,
            )

        s = jnp.maximum(jnp.max(jnp.abs(out_ref[:, :])), 1e-30)
        scale_s[:, :] = jnp.zeros((8, 128), jnp.float32) + s
        pl.semaphore_wait(barrier_sem, 1)

        s_rdma = pltpu.make_async_remote_copy(
            src_ref=scale_s, dst_ref=scale_r,
            send_sem=ssend_sem, recv_sem=srecv_sem,
            device_id=nbr, device_id_type=pl.DeviceIdType.MESH,
        )
        s_rdma.start()

        inv = 127.0 / s
        rdmas = []
        for c in range(NC):
            rows = pl.ds(c * mc, mc)
            send_q[c] = jnp.round(out_ref[rows, :] * inv).astype(jnp.int8)
            rdma = pltpu.make_async_remote_copy(
                src_ref=send_q.at[c],
                dst_ref=recv_q.at[c],
                send_sem=send_sems.at[c],
                recv_sem=recv_sems.at[c],
                device_id=nbr,
                device_id_type=pl.DeviceIdType.MESH,
            )
            rdma.start()
            rdmas.append(rdma)

        s_rdma.wait_recv()
        dq = scale_r[0, 0] * (1.0 / 127.0)
        for c in range(NC):
            rows = pl.ds(c * mc, mc)
            rdmas[c].wait_recv()
            out_ref[rows, :] = out_ref[rows, :] + recv_q[c].astype(jnp.float32) * dq
        s_rdma.wait_send()
        for c in range(NC):
            rdmas[c].wait_send()

    return pl.pallas_call(
        body,
        out_shape=jax.ShapeDtypeStruct((m, n), jnp.float32),
        in_specs=[
            pl.BlockSpec(memory_space=pltpu.VMEM),
            pl.BlockSpec(memory_space=pltpu.VMEM),
        ],
        out_specs=pl.BlockSpec(memory_space=pltpu.VMEM),
        scratch_shapes=[
            pltpu.VMEM((NC, mc, n), jnp.int8),
            pltpu.VMEM((NC, mc, n), jnp.int8),
            pltpu.VMEM((8, 128), jnp.float32),
            pltpu.VMEM((8, 128), jnp.float32),
            pltpu.SemaphoreType.DMA((NC,)),
            pltpu.SemaphoreType.DMA((NC,)),
            pltpu.SemaphoreType.DMA,
            pltpu.SemaphoreType.DMA,
        ],
        compiler_params=pltpu.CompilerParams(collective_id=0),
    )(A, B)
